# baseline (speedup 1.0000x reference)
import functools

import jax
import jax.numpy as jnp
from jax import lax
from jax.experimental import pallas as pl
from jax.experimental.pallas import tpu as pltpu
from jax.experimental.pallas import tpu_sc as plsc

_B = 1024
_LANES = 16
_NUM_CORES = 2
_NUM_SUBCORES = 16
_NW = _NUM_CORES * _NUM_SUBCORES
_BPW = _B // _NW


def _emb_body(idx_hbm, w_hbm, out_hbm, idx_v, out_v, w_v, sem):
    wid = lax.axis_index("s") * _NUM_CORES + lax.axis_index("c")
    base = wid * _BPW
    cp_w = pltpu.async_copy(w_hbm, w_v.at[pl.ds(0, 1)], sem)
    cp_i = pltpu.async_copy(idx_hbm.at[pl.ds(base, _BPW)], idx_v, sem)
    cp_w.wait()
    cp_i.wait()
    w_vec = w_v[...]
    for j in range(_BPW // _LANES):
        idx = idx_v[pl.ds(j * _LANES, _LANES)]
        out_v[pl.ds(j * _LANES, _LANES)] = w_vec.at[idx].get(
            mode="promise_in_bounds"
        )
    pltpu.sync_copy(out_v, out_hbm.at[pl.ds(base, _BPW)])


_emb = functools.partial(
    pl.kernel,
    mesh=plsc.VectorSubcoreMesh(
        core_axis_name="c",
        subcore_axis_name="s",
        num_cores=_NUM_CORES,
        num_subcores=_NUM_SUBCORES,
    ),
    out_type=jax.ShapeDtypeStruct((_B,), jnp.float32),
    scratch_types=[
        pltpu.VMEM((_BPW,), jnp.int32),
        pltpu.VMEM((_BPW,), jnp.float32),
        pltpu.VMEM((_LANES,), jnp.float32),
        pltpu.SemaphoreType.DMA,
    ],
)(_emb_body)


@jax.jit
def kernel(indices, weight):
    out = _emb(indices, jnp.reshape(weight, (-1,)))
    return jnp.reshape(out, (_B, weight.shape[1]))


# speedup vs baseline: 1.0979x; 1.0979x over previous
import functools

import jax
import jax.numpy as jnp
from jax import lax
from jax.experimental import pallas as pl
from jax.experimental.pallas import tpu as pltpu
from jax.experimental.pallas import tpu_sc as plsc

_B = 1024
_LANES = 16
_NUM_CORES = 1
_NUM_SUBCORES = 16
_NW = _NUM_CORES * _NUM_SUBCORES
_BPW = _B // _NW


def _emb_body(idx_hbm, w_hbm, out_hbm, idx_v, out_v, w_v, sem):
    wid = lax.axis_index("s") * _NUM_CORES + lax.axis_index("c")
    base = wid * _BPW
    cp_w = pltpu.async_copy(w_hbm, w_v.at[pl.ds(0, 1)], sem)
    cp_i = pltpu.async_copy(idx_hbm.at[pl.ds(base, _BPW)], idx_v, sem)
    cp_w.wait()
    cp_i.wait()
    w_vec = w_v[...]
    for j in range(_BPW // _LANES):
        idx = idx_v[pl.ds(j * _LANES, _LANES)]
        out_v[pl.ds(j * _LANES, _LANES)] = w_vec.at[idx].get(
            mode="promise_in_bounds"
        )
    pltpu.sync_copy(out_v, out_hbm.at[pl.ds(base, _BPW)])


_emb = functools.partial(
    pl.kernel,
    mesh=plsc.VectorSubcoreMesh(
        core_axis_name="c",
        subcore_axis_name="s",
        num_cores=_NUM_CORES,
        num_subcores=_NUM_SUBCORES,
    ),
    out_type=jax.ShapeDtypeStruct((_B,), jnp.float32),
    scratch_types=[
        pltpu.VMEM((_BPW,), jnp.int32),
        pltpu.VMEM((_BPW,), jnp.float32),
        pltpu.VMEM((_LANES,), jnp.float32),
        pltpu.SemaphoreType.DMA,
    ],
)(_emb_body)


@jax.jit
def kernel(indices, weight):
    out = _emb(indices, jnp.reshape(weight, (-1,)))
    return jnp.reshape(out, (_B, weight.shape[1]))
